# Initial kernel scaffold; baseline (speedup 1.0000x reference)
import functools

import jax
import jax.numpy as jnp
from jax import lax
from jax.experimental import pallas as pl
from jax.experimental.pallas import tpu as pltpu
from jax.experimental.pallas import tpu_sc as plsc

N = 10000
E = 320000
H = 128
G = 64
NG = 2
NC = 2
NS = 16
K = 128
C = 158
TPT = C * K
EPAD = NS * TPT
NLOC = 10240
IPR = N // NS

BM = 800
NB = 400
NBLK = N // NB

_f32 = jnp.float32


def _make_agg():
    mesh = plsc.VectorSubcoreMesh(core_axis_name="c", subcore_axis_name="s")

    @functools.partial(
        pl.kernel,
        out_type=jax.ShapeDtypeStruct((NG * N, H), _f32),
        mesh=mesh,
        scratch_types=[
            pltpu.VMEM((K,), jnp.int32),
            pltpu.VMEM((K,), jnp.int32),
            pltpu.VMEM((K,), jnp.int32),
            pltpu.VMEM((K,), jnp.int32),
            pltpu.VMEM((K, H), _f32),
            pltpu.VMEM((K, H), _f32),
            pltpu.VMEM_SHARED((NLOC, H), _f32),
            pltpu.SemaphoreType.DMA,
            pltpu.SemaphoreType.DMA,
        ],
    )
    def agg(x_hbm, src_hbm, dst_hbm, out_hbm,
            sidx0, sidx1, didx0, didx1, rows0, rows1, agg_sh, sem0, sem1):
        c = lax.axis_index("c")
        s = lax.axis_index("s")
        pltpu.sync_copy(x_hbm.at[pl.ds(c * N + s * IPR, IPR)],
                        agg_sh.at[pl.ds(s * IPR, IPR)])
        plsc.subcore_barrier()

        row0 = (c * NS + s) * C
        sidx = (sidx0, sidx1)
        didx = (didx0, didx1)
        rows = (rows0, rows1)
        sem = (sem0, sem1)

        def load_and_fire(jj, b):
            pltpu.sync_copy(src_hbm.at[row0 + jj], sidx[b])
            pltpu.sync_copy(dst_hbm.at[row0 + jj], didx[b])
            pltpu.async_copy(x_hbm.at[sidx[b]], rows[b], sem[b])

        load_and_fire(0, 0)
        load_and_fire(1, 1)

        def body(i, carry):
            j = 2 * i
            for b in range(2):
                jj = j + b
                pltpu.make_async_copy(x_hbm.at[sidx[b]], rows[b], sem[b]).wait()
                pltpu.sync_copy(rows[b], agg_sh.at[didx[b]], add=True)

                @pl.when(jj + 2 < C)
                def _():
                    load_and_fire(jj + 2, b)
            return carry

        lax.fori_loop(0, C // 2, body, 0)
        plsc.subcore_barrier()
        pltpu.sync_copy(agg_sh.at[pl.ds(s * IPR, IPR)],
                        out_hbm.at[pl.ds(c * N + s * IPR, IPR)])

    return agg


_agg_call = _make_agg()


def _pre_call(x, w, b):
    def body(x_ref, w_ref, b_ref, out_ref):
        out_ref[...] = jnp.dot(x_ref[...], w_ref[...],
                               preferred_element_type=_f32) + b_ref[...]

    m = x.shape[0]
    return pl.pallas_call(
        body,
        grid=(m // BM,),
        in_specs=[
            pl.BlockSpec((BM, H), lambda i: (i, 0)),
            pl.BlockSpec((H, H), lambda i: (0, 0)),
            pl.BlockSpec((1, H), lambda i: (0, 0)),
        ],
        out_specs=pl.BlockSpec((BM, H), lambda i: (i, 0)),
        out_shape=jax.ShapeDtypeStruct((m, H), _f32),
    )(x, w, b)


def _mlp_call(hagg, w1, b1, w2, b2, xres=None):
    has_res = xres is not None

    def body(*refs):
        if has_res:
            h_ref, r_ref, w1r, b1r, w2r, b2r, out_ref = refs
        else:
            h_ref, w1r, b1r, w2r, b2r, out_ref = refs
        t = jnp.dot(h_ref[...], w1r[...], preferred_element_type=_f32) + b1r[...]
        t = jnp.maximum(t, 0.0)
        y = jnp.dot(t, w2r[...], preferred_element_type=_f32) + b2r[...]
        if has_res:
            y = y + r_ref[...]
        out_ref[...] = jnp.maximum(y, 0.0)

    m = hagg.shape[0]
    in_specs = [pl.BlockSpec((BM, H), lambda i: (i, 0))]
    args = [hagg]
    if has_res:
        in_specs.append(pl.BlockSpec((BM, H), lambda i: (i, 0)))
        args.append(xres)
    in_specs += [
        pl.BlockSpec((H, H), lambda i: (0, 0)),
        pl.BlockSpec((1, H), lambda i: (0, 0)),
        pl.BlockSpec((H, H), lambda i: (0, 0)),
        pl.BlockSpec((1, H), lambda i: (0, 0)),
    ]
    args += [w1, b1, w2, b2]
    return pl.pallas_call(
        body,
        grid=(m // BM,),
        in_specs=in_specs,
        out_specs=pl.BlockSpec((BM, H), lambda i: (i, 0)),
        out_shape=jax.ShapeDtypeStruct((m, H), _f32),
    )(*args)


def _pool_call(x0, x1, x2, x3, batch3d, wp1, bp1, wp2, bp2):
    def body(x0r, x1r, x2r, x3r, br, wp1r, bp1r, wp2r, bp2r,
             out_ref, acc, outg):
        g = pl.program_id(0)
        j = pl.program_id(1)

        @pl.when(j == 0)
        def _():
            acc[...] = jnp.zeros_like(acc)

        bvec = br[0, 0, :]
        onehot = (bvec[None, :] ==
                  lax.broadcasted_iota(jnp.int32, (G, NB), 0)).astype(_f32)
        emb = jnp.concatenate(
            [x0r[...], x1r[...], x2r[...], x3r[...]], axis=1)
        acc[...] += jnp.dot(onehot, emb, preferred_element_type=_f32)

        @pl.when(j == NBLK - 1)
        def _():
            t = jnp.dot(acc[...], wp1r[...], preferred_element_type=_f32)
            t = jnp.maximum(t + bp1r[...], 0.0)
            o = jnp.dot(t, wp2r[...], preferred_element_type=_f32) + bp2r[...]

            @pl.when(g == 0)
            def _():
                outg[...] = o

            @pl.when(g == 1)
            def _():
                d = outg[...] - o
                out_ref[...] = jnp.sqrt(jnp.sum(d * d, axis=1))[None, :]

    xspec = pl.BlockSpec((NB, H), lambda g, j: (g * NBLK + j, 0))
    return pl.pallas_call(
        body,
        grid=(NG, NBLK),
        in_specs=[
            xspec, xspec, xspec, xspec,
            pl.BlockSpec((1, 1, NB), lambda g, j: (g * NBLK + j, 0, 0)),
            pl.BlockSpec((4 * H, H), lambda g, j: (0, 0)),
            pl.BlockSpec((1, H), lambda g, j: (0, 0)),
            pl.BlockSpec((H, H), lambda g, j: (0, 0)),
            pl.BlockSpec((1, H), lambda g, j: (0, 0)),
        ],
        out_specs=pl.BlockSpec((1, G), lambda g, j: (0, 0)),
        out_shape=jax.ShapeDtypeStruct((1, G), _f32),
        scratch_shapes=[
            pltpu.VMEM((G, 4 * H), _f32),
            pltpu.VMEM((G, H), _f32),
        ],
    )(x0, x1, x2, x3, batch3d, wp1, bp1, wp2, bp2)


def _prep_edges(edge_index, goff):
    pad = EPAD - E
    src = jnp.concatenate([
        edge_index[0] + goff,
        (jnp.arange(pad, dtype=jnp.int32) * 37) % N + goff,
    ])
    dst = jnp.concatenate([
        edge_index[1],
        N + (jnp.arange(pad, dtype=jnp.int32) % (NLOC - N)),
    ])
    return src.reshape(NS * C, K), dst.reshape(NS * C, K)


def kernel(g_x, g_edge_index, g_batch, h_x, h_edge_index, h_batch,
           W_pre, b_pre, W1, b1, W2, b2, Wp1, bp1, Wp2, bp2):
    X = jnp.concatenate([g_x, h_x], axis=0)
    sg, dg = _prep_edges(g_edge_index, 0)
    sh, dh = _prep_edges(h_edge_index, N)
    src_all = jnp.concatenate([sg, sh], axis=0)
    dst_all = jnp.concatenate([dg, dh], axis=0)

    x0 = _pre_call(X, W_pre, b_pre.reshape(1, H))
    x = x0
    xs = [x0]
    for i in range(3):
        hagg = _agg_call(x, src_all, dst_all)
        x = _mlp_call(hagg, W1[i], b1[i].reshape(1, H),
                      W2[i], b2[i].reshape(1, H),
                      xres=x0 if i == 1 else None)
        xs.append(x)

    batch3d = jnp.concatenate([g_batch, h_batch]).reshape(NG * NBLK, 1, NB)
    out = _pool_call(xs[0], xs[1], xs[2], xs[3], batch3d,
                     Wp1, bp1.reshape(1, H), Wp2, bp2.reshape(1, H))
    return out.reshape(G)


# speedup vs baseline: 5.8505x; 5.8505x over previous
import functools

import jax
import jax.numpy as jnp
from jax import lax
from jax.experimental import pallas as pl
from jax.experimental.pallas import tpu as pltpu
from jax.experimental.pallas import tpu_sc as plsc

N = 10000
E = 320000
H = 128
G = 64
NG = 2
NC = 2
NS = 16
K = 128
C = 158
TPT = C * K
EPAD = NS * TPT
NLOC = 10240
IPR = 624
REM = N - NS * IPR

BM = 800
NB = 400
NBLK = N // NB

_f32 = jnp.float32


def _make_agg():
    mesh = plsc.VectorSubcoreMesh(core_axis_name="c", subcore_axis_name="s")

    @functools.partial(
        pl.kernel,
        out_type=jax.ShapeDtypeStruct((NG * N, H), _f32),
        mesh=mesh,
        scratch_types=[
            pltpu.VMEM((K,), jnp.int32),
            pltpu.VMEM((K,), jnp.int32),
            pltpu.VMEM((K,), jnp.int32),
            pltpu.VMEM((K,), jnp.int32),
            pltpu.VMEM((K, H), _f32),
            pltpu.VMEM((K, H), _f32),
            pltpu.VMEM_SHARED((NLOC, H), _f32),
            pltpu.SemaphoreType.DMA,
            pltpu.SemaphoreType.DMA,
        ],
    )
    def agg(x_hbm, src_hbm, dst_hbm, out_hbm,
            sidx0, sidx1, didx0, didx1, rows0, rows1, agg_sh, sem0, sem1):
        c = lax.axis_index("c")
        s = lax.axis_index("s")
        pltpu.sync_copy(x_hbm.at[pl.ds(c * N + s * IPR, IPR)],
                        agg_sh.at[pl.ds(s * IPR, IPR)])

        @pl.when(s == 0)
        def _():
            pltpu.sync_copy(x_hbm.at[pl.ds(c * N + NS * IPR, REM)],
                            agg_sh.at[pl.ds(NS * IPR, REM)])

        plsc.subcore_barrier()

        row0 = (c * NS + s) * C
        sidx = (sidx0, sidx1)
        didx = (didx0, didx1)
        rows = (rows0, rows1)
        sem = (sem0, sem1)

        def load_and_fire(jj, b):
            pltpu.sync_copy(src_hbm.at[row0 + jj], sidx[b])
            pltpu.sync_copy(dst_hbm.at[row0 + jj], didx[b])
            pltpu.async_copy(x_hbm.at[sidx[b]], rows[b], sem[b])

        load_and_fire(0, 0)
        load_and_fire(1, 1)

        def body(i, carry):
            j = 2 * i
            for b in range(2):
                jj = j + b
                pltpu.make_async_copy(x_hbm.at[sidx[b]], rows[b], sem[b]).wait()
                pltpu.sync_copy(rows[b], agg_sh.at[didx[b]], add=True)

                @pl.when(jj + 2 < C)
                def _():
                    load_and_fire(jj + 2, b)
            return carry

        lax.fori_loop(0, C // 2, body, 0)
        plsc.subcore_barrier()
        pltpu.sync_copy(agg_sh.at[pl.ds(s * IPR, IPR)],
                        out_hbm.at[pl.ds(c * N + s * IPR, IPR)])

        @pl.when(s == 0)
        def _():
            pltpu.sync_copy(agg_sh.at[pl.ds(NS * IPR, REM)],
                            out_hbm.at[pl.ds(c * N + NS * IPR, REM)])

    return agg


_agg_call = _make_agg()


def _pre_call(x, w, b):
    def body(x_ref, w_ref, b_ref, out_ref):
        out_ref[...] = jnp.dot(x_ref[...], w_ref[...],
                               preferred_element_type=_f32) + b_ref[...]

    m = x.shape[0]
    return pl.pallas_call(
        body,
        grid=(m // BM,),
        in_specs=[
            pl.BlockSpec((BM, H), lambda i: (i, 0)),
            pl.BlockSpec((H, H), lambda i: (0, 0)),
            pl.BlockSpec((1, H), lambda i: (0, 0)),
        ],
        out_specs=pl.BlockSpec((BM, H), lambda i: (i, 0)),
        out_shape=jax.ShapeDtypeStruct((m, H), _f32),
    )(x, w, b)


def _mlp_call(hagg, w1, b1, w2, b2, xres=None):
    has_res = xres is not None

    def body(*refs):
        if has_res:
            h_ref, r_ref, w1r, b1r, w2r, b2r, out_ref = refs
        else:
            h_ref, w1r, b1r, w2r, b2r, out_ref = refs
        t = jnp.dot(h_ref[...], w1r[...], preferred_element_type=_f32) + b1r[...]
        t = jnp.maximum(t, 0.0)
        y = jnp.dot(t, w2r[...], preferred_element_type=_f32) + b2r[...]
        if has_res:
            y = y + r_ref[...]
        out_ref[...] = jnp.maximum(y, 0.0)

    m = hagg.shape[0]
    in_specs = [pl.BlockSpec((BM, H), lambda i: (i, 0))]
    args = [hagg]
    if has_res:
        in_specs.append(pl.BlockSpec((BM, H), lambda i: (i, 0)))
        args.append(xres)
    in_specs += [
        pl.BlockSpec((H, H), lambda i: (0, 0)),
        pl.BlockSpec((1, H), lambda i: (0, 0)),
        pl.BlockSpec((H, H), lambda i: (0, 0)),
        pl.BlockSpec((1, H), lambda i: (0, 0)),
    ]
    args += [w1, b1, w2, b2]
    return pl.pallas_call(
        body,
        grid=(m // BM,),
        in_specs=in_specs,
        out_specs=pl.BlockSpec((BM, H), lambda i: (i, 0)),
        out_shape=jax.ShapeDtypeStruct((m, H), _f32),
    )(*args)


def _pool_call(x0, x1, x2, x3, batch3d, wp1, bp1, wp2, bp2):
    def body(x0r, x1r, x2r, x3r, br, wp1r, bp1r, wp2r, bp2r,
             out_ref, acc, outg):
        g = pl.program_id(0)
        j = pl.program_id(1)

        @pl.when(j == 0)
        def _():
            acc[...] = jnp.zeros_like(acc)

        bvec = br[0, 0, :]
        onehot = (bvec[None, :] ==
                  lax.broadcasted_iota(jnp.int32, (G, NB), 0)).astype(_f32)
        emb = jnp.concatenate(
            [x0r[...], x1r[...], x2r[...], x3r[...]], axis=1)
        acc[...] += jnp.dot(onehot, emb, preferred_element_type=_f32)

        @pl.when(j == NBLK - 1)
        def _():
            t = jnp.dot(acc[...], wp1r[...], preferred_element_type=_f32)
            t = jnp.maximum(t + bp1r[...], 0.0)
            o = jnp.dot(t, wp2r[...], preferred_element_type=_f32) + bp2r[...]

            @pl.when(g == 0)
            def _():
                outg[...] = o

            @pl.when(g == 1)
            def _():
                d = outg[...] - o
                out_ref[...] = jnp.sqrt(jnp.sum(d * d, axis=1))[None, :]

    xspec = pl.BlockSpec((NB, H), lambda g, j: (g * NBLK + j, 0))
    return pl.pallas_call(
        body,
        grid=(NG, NBLK),
        in_specs=[
            xspec, xspec, xspec, xspec,
            pl.BlockSpec((1, 1, NB), lambda g, j: (g * NBLK + j, 0, 0)),
            pl.BlockSpec((4 * H, H), lambda g, j: (0, 0)),
            pl.BlockSpec((1, H), lambda g, j: (0, 0)),
            pl.BlockSpec((H, H), lambda g, j: (0, 0)),
            pl.BlockSpec((1, H), lambda g, j: (0, 0)),
        ],
        out_specs=pl.BlockSpec((1, G), lambda g, j: (0, 0)),
        out_shape=jax.ShapeDtypeStruct((1, G), _f32),
        scratch_shapes=[
            pltpu.VMEM((G, 4 * H), _f32),
            pltpu.VMEM((G, H), _f32),
        ],
    )(x0, x1, x2, x3, batch3d, wp1, bp1, wp2, bp2)


def _prep_edges(edge_index, goff):
    pad = EPAD - E
    src = jnp.concatenate([
        edge_index[0] + goff,
        (jnp.arange(pad, dtype=jnp.int32) * 37) % N + goff,
    ])
    dst = jnp.concatenate([
        edge_index[1],
        N + (jnp.arange(pad, dtype=jnp.int32) % (NLOC - N)),
    ])
    return src.reshape(NS * C, K), dst.reshape(NS * C, K)


def kernel(g_x, g_edge_index, g_batch, h_x, h_edge_index, h_batch,
           W_pre, b_pre, W1, b1, W2, b2, Wp1, bp1, Wp2, bp2):
    X = jnp.concatenate([g_x, h_x], axis=0)
    sg, dg = _prep_edges(g_edge_index, 0)
    sh, dh = _prep_edges(h_edge_index, N)
    src_all = jnp.concatenate([sg, sh], axis=0)
    dst_all = jnp.concatenate([dg, dh], axis=0)

    x0 = _pre_call(X, W_pre, b_pre.reshape(1, H))
    x = x0
    xs = [x0]
    for i in range(3):
        hagg = _agg_call(x, src_all, dst_all)
        x = _mlp_call(hagg, W1[i], b1[i].reshape(1, H),
                      W2[i], b2[i].reshape(1, H),
                      xres=x0 if i == 1 else None)
        xs.append(x)

    batch3d = jnp.concatenate([g_batch, h_batch]).reshape(NG * NBLK, 1, NB)
    out = _pool_call(xs[0], xs[1], xs[2], xs[3], batch3d,
                     Wp1, bp1.reshape(1, H), Wp2, bp2.reshape(1, H))
    return out.reshape(G)


# speedup vs baseline: 5.8612x; 1.0018x over previous
import functools

import jax
import jax.numpy as jnp
from jax import lax
from jax.experimental import pallas as pl
from jax.experimental.pallas import tpu as pltpu
from jax.experimental.pallas import tpu_sc as plsc

N = 10000
E = 320000
H = 128
G = 64
NG = 2
NC = 2
NS = 16
K = 128
C = 158
TPT = C * K
EPAD = NS * TPT
NLOC = 10240
IPR = 624
REM = N - NS * IPR

NBUF = 2
D = 1

BM = 800
NB = 400
NBLK = N // NB

_f32 = jnp.float32


def _make_agg():
    mesh = plsc.VectorSubcoreMesh(core_axis_name="c", subcore_axis_name="s",
                                  num_cores=NC, num_subcores=NS)

    @functools.partial(
        pl.kernel,
        out_type=jax.ShapeDtypeStruct((NG * N, H), _f32),
        mesh=mesh,
        scratch_types=(
            [pltpu.VMEM((K,), jnp.int32) for _ in range(2 * NBUF)]
            + [pltpu.VMEM((K, H), _f32) for _ in range(NBUF)]
            + [pltpu.VMEM_SHARED((NLOC, H), _f32)]
            + [pltpu.SemaphoreType.DMA for _ in range(2 * NBUF)]
        ),
    )
    def agg(x_hbm, src_hbm, dst_hbm, out_hbm, *scratch):
        sidx = scratch[0:NBUF]
        didx = scratch[NBUF:2 * NBUF]
        rows = scratch[2 * NBUF:3 * NBUF]
        agg_sh = scratch[3 * NBUF]
        gsem = scratch[3 * NBUF + 1:3 * NBUF + 1 + NBUF]
        ssem = scratch[3 * NBUF + 1 + NBUF:3 * NBUF + 1 + 2 * NBUF]
        c = lax.axis_index("c")
        s = lax.axis_index("s")
        pltpu.sync_copy(x_hbm.at[pl.ds(c * N + s * IPR, IPR)],
                        agg_sh.at[pl.ds(s * IPR, IPR)])

        @pl.when(s == 0)
        def _():
            pltpu.sync_copy(x_hbm.at[pl.ds(c * N + NS * IPR, REM)],
                            agg_sh.at[pl.ds(NS * IPR, REM)])

        plsc.subcore_barrier()

        row0 = (c * NS + s) * C

        def body(i, carry):
            for b in range(NBUF):
                jj = i * NBUF + b

                @pl.when(jnp.logical_and(jj >= NBUF, jj < C))
                def _():
                    pltpu.make_async_copy(
                        rows[b], agg_sh.at[didx[b]], ssem[b]).wait()

                @pl.when(jj < C)
                def _():
                    pltpu.sync_copy(src_hbm.at[row0 + jj], sidx[b])
                    pltpu.sync_copy(dst_hbm.at[row0 + jj], didx[b])
                    pltpu.async_copy(x_hbm.at[sidx[b]], rows[b], gsem[b])

                bd = (b - D) % NBUF

                @pl.when(jnp.logical_and(jj >= D, jj < C + D))
                def _():
                    pltpu.make_async_copy(
                        x_hbm.at[sidx[bd]], rows[bd], gsem[bd]).wait()
                    pltpu.async_copy(rows[bd], agg_sh.at[didx[bd]],
                                     ssem[bd], add=True)
            return carry

        lax.fori_loop(0, (C + D + NBUF - 1) // NBUF, body, 0)
        for jj in range(C - NBUF, C):
            b = jj % NBUF
            pltpu.make_async_copy(rows[b], agg_sh.at[didx[b]], ssem[b]).wait()
        plsc.subcore_barrier()
        pltpu.sync_copy(agg_sh.at[pl.ds(s * IPR, IPR)],
                        out_hbm.at[pl.ds(c * N + s * IPR, IPR)])

        @pl.when(s == 0)
        def _():
            pltpu.sync_copy(agg_sh.at[pl.ds(NS * IPR, REM)],
                            out_hbm.at[pl.ds(c * N + NS * IPR, REM)])

    return agg


_agg_call = _make_agg()


def _pre_call(x, w, b):
    def body(x_ref, w_ref, b_ref, out_ref):
        out_ref[...] = jnp.dot(x_ref[...], w_ref[...],
                               preferred_element_type=_f32) + b_ref[...]

    m = x.shape[0]
    return pl.pallas_call(
        body,
        grid=(m // BM,),
        in_specs=[
            pl.BlockSpec((BM, H), lambda i: (i, 0)),
            pl.BlockSpec((H, H), lambda i: (0, 0)),
            pl.BlockSpec((1, H), lambda i: (0, 0)),
        ],
        out_specs=pl.BlockSpec((BM, H), lambda i: (i, 0)),
        out_shape=jax.ShapeDtypeStruct((m, H), _f32),
    )(x, w, b)


def _mlp_call(hagg, w1, b1, w2, b2, xres=None):
    has_res = xres is not None

    def body(*refs):
        if has_res:
            h_ref, r_ref, w1r, b1r, w2r, b2r, out_ref = refs
        else:
            h_ref, w1r, b1r, w2r, b2r, out_ref = refs
        t = jnp.dot(h_ref[...], w1r[...], preferred_element_type=_f32) + b1r[...]
        t = jnp.maximum(t, 0.0)
        y = jnp.dot(t, w2r[...], preferred_element_type=_f32) + b2r[...]
        if has_res:
            y = y + r_ref[...]
        out_ref[...] = jnp.maximum(y, 0.0)

    m = hagg.shape[0]
    in_specs = [pl.BlockSpec((BM, H), lambda i: (i, 0))]
    args = [hagg]
    if has_res:
        in_specs.append(pl.BlockSpec((BM, H), lambda i: (i, 0)))
        args.append(xres)
    in_specs += [
        pl.BlockSpec((H, H), lambda i: (0, 0)),
        pl.BlockSpec((1, H), lambda i: (0, 0)),
        pl.BlockSpec((H, H), lambda i: (0, 0)),
        pl.BlockSpec((1, H), lambda i: (0, 0)),
    ]
    args += [w1, b1, w2, b2]
    return pl.pallas_call(
        body,
        grid=(m // BM,),
        in_specs=in_specs,
        out_specs=pl.BlockSpec((BM, H), lambda i: (i, 0)),
        out_shape=jax.ShapeDtypeStruct((m, H), _f32),
    )(*args)


def _pool_call(x0, x1, x2, x3, batch3d, wp1, bp1, wp2, bp2):
    def body(x0r, x1r, x2r, x3r, br, wp1r, bp1r, wp2r, bp2r,
             out_ref, acc, outg):
        g = pl.program_id(0)
        j = pl.program_id(1)

        @pl.when(j == 0)
        def _():
            acc[...] = jnp.zeros_like(acc)

        bvec = br[0, 0, :]
        onehot = (bvec[None, :] ==
                  lax.broadcasted_iota(jnp.int32, (G, NB), 0)).astype(_f32)
        emb = jnp.concatenate(
            [x0r[...], x1r[...], x2r[...], x3r[...]], axis=1)
        acc[...] += jnp.dot(onehot, emb, preferred_element_type=_f32)

        @pl.when(j == NBLK - 1)
        def _():
            t = jnp.dot(acc[...], wp1r[...], preferred_element_type=_f32)
            t = jnp.maximum(t + bp1r[...], 0.0)
            o = jnp.dot(t, wp2r[...], preferred_element_type=_f32) + bp2r[...]

            @pl.when(g == 0)
            def _():
                outg[...] = o

            @pl.when(g == 1)
            def _():
                d = outg[...] - o
                out_ref[...] = jnp.sqrt(jnp.sum(d * d, axis=1))[None, :]

    xspec = pl.BlockSpec((NB, H), lambda g, j: (g * NBLK + j, 0))
    return pl.pallas_call(
        body,
        grid=(NG, NBLK),
        in_specs=[
            xspec, xspec, xspec, xspec,
            pl.BlockSpec((1, 1, NB), lambda g, j: (g * NBLK + j, 0, 0)),
            pl.BlockSpec((4 * H, H), lambda g, j: (0, 0)),
            pl.BlockSpec((1, H), lambda g, j: (0, 0)),
            pl.BlockSpec((H, H), lambda g, j: (0, 0)),
            pl.BlockSpec((1, H), lambda g, j: (0, 0)),
        ],
        out_specs=pl.BlockSpec((1, G), lambda g, j: (0, 0)),
        out_shape=jax.ShapeDtypeStruct((1, G), _f32),
        scratch_shapes=[
            pltpu.VMEM((G, 4 * H), _f32),
            pltpu.VMEM((G, H), _f32),
        ],
    )(x0, x1, x2, x3, batch3d, wp1, bp1, wp2, bp2)


def _prep_edges(edge_index, goff):
    pad = EPAD - E
    src = jnp.concatenate([
        edge_index[0] + goff,
        (jnp.arange(pad, dtype=jnp.int32) * 37) % N + goff,
    ])
    dst = jnp.concatenate([
        edge_index[1],
        N + (jnp.arange(pad, dtype=jnp.int32) % (NLOC - N)),
    ])
    return src.reshape(NS * C, K), dst.reshape(NS * C, K)


def kernel(g_x, g_edge_index, g_batch, h_x, h_edge_index, h_batch,
           W_pre, b_pre, W1, b1, W2, b2, Wp1, bp1, Wp2, bp2):
    X = jnp.concatenate([g_x, h_x], axis=0)
    sg, dg = _prep_edges(g_edge_index, 0)
    sh, dh = _prep_edges(h_edge_index, N)
    src_all = jnp.concatenate([sg, sh], axis=0)
    dst_all = jnp.concatenate([dg, dh], axis=0)

    x0 = _pre_call(X, W_pre, b_pre.reshape(1, H))
    x = x0
    xs = [x0]
    for i in range(3):
        hagg = _agg_call(x, src_all, dst_all)
        x = _mlp_call(hagg, W1[i], b1[i].reshape(1, H),
                      W2[i], b2[i].reshape(1, H),
                      xres=x0 if i == 1 else None)
        xs.append(x)

    batch3d = jnp.concatenate([g_batch, h_batch]).reshape(NG * NBLK, 1, NB)
    out = _pool_call(xs[0], xs[1], xs[2], xs[3], batch3d,
                     Wp1, bp1.reshape(1, H), Wp2, bp2.reshape(1, H))
    return out.reshape(G)


# speedup vs baseline: 7.2383x; 1.2350x over previous
import functools

import jax
import jax.numpy as jnp
from jax import lax
from jax.experimental import pallas as pl
from jax.experimental.pallas import tpu as pltpu
from jax.experimental.pallas import tpu_sc as plsc

N = 10000
E = 320000
H = 128
G = 64
NG = 2
NC = 2
NS = 16
K = 128
C = 158
TPT = C * K
EPAD = NS * TPT
NLOC = 10048
IPR = 624
REM = N - NS * IPR

NBUF = 3
D = 1

BM = 800
NB = 400
NBLK = N // NB

_f32 = jnp.float32


def _make_agg():
    mesh = plsc.VectorSubcoreMesh(core_axis_name="c", subcore_axis_name="s",
                                  num_cores=NC, num_subcores=NS)

    @functools.partial(
        pl.kernel,
        out_type=jax.ShapeDtypeStruct((NG * N, H), _f32),
        mesh=mesh,
        scratch_types=(
            [pltpu.VMEM((K,), jnp.int32) for _ in range(2 * NBUF)]
            + [pltpu.VMEM((K, H), _f32) for _ in range(NBUF)]
            + [pltpu.VMEM_SHARED((NLOC, H), _f32)]
            + [pltpu.SemaphoreType.DMA for _ in range(2 * NBUF)]
        ),
    )
    def agg(x_hbm, src_hbm, dst_hbm, out_hbm, *scratch):
        sidx = scratch[0:NBUF]
        didx = scratch[NBUF:2 * NBUF]
        rows = scratch[2 * NBUF:3 * NBUF]
        agg_sh = scratch[3 * NBUF]
        gsem = scratch[3 * NBUF + 1:3 * NBUF + 1 + NBUF]
        ssem = scratch[3 * NBUF + 1 + NBUF:3 * NBUF + 1 + 2 * NBUF]
        c = lax.axis_index("c")
        s = lax.axis_index("s")
        pltpu.sync_copy(x_hbm.at[pl.ds(c * N + s * IPR, IPR)],
                        agg_sh.at[pl.ds(s * IPR, IPR)])

        @pl.when(s == 0)
        def _():
            pltpu.sync_copy(x_hbm.at[pl.ds(c * N + NS * IPR, REM)],
                            agg_sh.at[pl.ds(NS * IPR, REM)])

        plsc.subcore_barrier()

        row0 = (c * NS + s) * C

        def body(i, carry):
            for b in range(NBUF):
                jj = i * NBUF + b

                @pl.when(jnp.logical_and(jj >= NBUF, jj < C))
                def _():
                    pltpu.make_async_copy(
                        rows[b], agg_sh.at[didx[b]], ssem[b]).wait()

                @pl.when(jj < C)
                def _():
                    pltpu.sync_copy(src_hbm.at[row0 + jj], sidx[b])
                    pltpu.sync_copy(dst_hbm.at[row0 + jj], didx[b])
                    pltpu.async_copy(x_hbm.at[sidx[b]], rows[b], gsem[b])

                bd = (b - D) % NBUF

                @pl.when(jnp.logical_and(jj >= D, jj < C + D))
                def _():
                    pltpu.make_async_copy(
                        x_hbm.at[sidx[bd]], rows[bd], gsem[bd]).wait()
                    pltpu.async_copy(rows[bd], agg_sh.at[didx[bd]],
                                     ssem[bd], add=True)
            return carry

        lax.fori_loop(0, (C + D + NBUF - 1) // NBUF, body, 0)
        for jj in range(C - NBUF, C):
            b = jj % NBUF
            pltpu.make_async_copy(rows[b], agg_sh.at[didx[b]], ssem[b]).wait()
        plsc.subcore_barrier()
        pltpu.sync_copy(agg_sh.at[pl.ds(s * IPR, IPR)],
                        out_hbm.at[pl.ds(c * N + s * IPR, IPR)])

        @pl.when(s == 0)
        def _():
            pltpu.sync_copy(agg_sh.at[pl.ds(NS * IPR, REM)],
                            out_hbm.at[pl.ds(c * N + NS * IPR, REM)])

    return agg


_agg_call = _make_agg()


def _pre_call(x, w, b):
    def body(x_ref, w_ref, b_ref, out_ref):
        out_ref[...] = jnp.dot(x_ref[...], w_ref[...],
                               preferred_element_type=_f32) + b_ref[...]

    m = x.shape[0]
    return pl.pallas_call(
        body,
        grid=(m // BM,),
        in_specs=[
            pl.BlockSpec((BM, H), lambda i: (i, 0)),
            pl.BlockSpec((H, H), lambda i: (0, 0)),
            pl.BlockSpec((1, H), lambda i: (0, 0)),
        ],
        out_specs=pl.BlockSpec((BM, H), lambda i: (i, 0)),
        out_shape=jax.ShapeDtypeStruct((m, H), _f32),
    )(x, w, b)


def _mlp_call(hagg, w1, b1, w2, b2, xres=None):
    has_res = xres is not None

    def body(*refs):
        if has_res:
            h_ref, r_ref, w1r, b1r, w2r, b2r, out_ref = refs
        else:
            h_ref, w1r, b1r, w2r, b2r, out_ref = refs
        t = jnp.dot(h_ref[...], w1r[...], preferred_element_type=_f32) + b1r[...]
        t = jnp.maximum(t, 0.0)
        y = jnp.dot(t, w2r[...], preferred_element_type=_f32) + b2r[...]
        if has_res:
            y = y + r_ref[...]
        out_ref[...] = jnp.maximum(y, 0.0)

    m = hagg.shape[0]
    in_specs = [pl.BlockSpec((BM, H), lambda i: (i, 0))]
    args = [hagg]
    if has_res:
        in_specs.append(pl.BlockSpec((BM, H), lambda i: (i, 0)))
        args.append(xres)
    in_specs += [
        pl.BlockSpec((H, H), lambda i: (0, 0)),
        pl.BlockSpec((1, H), lambda i: (0, 0)),
        pl.BlockSpec((H, H), lambda i: (0, 0)),
        pl.BlockSpec((1, H), lambda i: (0, 0)),
    ]
    args += [w1, b1, w2, b2]
    return pl.pallas_call(
        body,
        grid=(m // BM,),
        in_specs=in_specs,
        out_specs=pl.BlockSpec((BM, H), lambda i: (i, 0)),
        out_shape=jax.ShapeDtypeStruct((m, H), _f32),
    )(*args)


def _pool_call(agg2, x0, x1, x2, batch3d, w1, b1, w2, b2,
               wp1, bp1, wp2, bp2):
    def body(aggr, x0r, x1r, x2r, br, w1r, b1r, w2r, b2r,
             wp1r, bp1r, wp2r, bp2r, out_ref, acc, outg):
        g = pl.program_id(0)
        j = pl.program_id(1)

        @pl.when(j == 0)
        def _():
            acc[...] = jnp.zeros_like(acc)

        t = jnp.dot(aggr[...], w1r[...], preferred_element_type=_f32)
        t = jnp.maximum(t + b1r[...], 0.0)
        x3 = jnp.dot(t, w2r[...], preferred_element_type=_f32) + b2r[...]
        x3 = jnp.maximum(x3, 0.0)

        bvec = br[0, 0, :]
        onehot = (bvec[None, :] ==
                  lax.broadcasted_iota(jnp.int32, (G, NB), 0)).astype(_f32)
        emb = jnp.concatenate(
            [x0r[...], x1r[...], x2r[...], x3], axis=1)
        acc[...] += jnp.dot(onehot, emb, preferred_element_type=_f32)

        @pl.when(j == NBLK - 1)
        def _():
            t = jnp.dot(acc[...], wp1r[...], preferred_element_type=_f32)
            t = jnp.maximum(t + bp1r[...], 0.0)
            o = jnp.dot(t, wp2r[...], preferred_element_type=_f32) + bp2r[...]

            @pl.when(g == 0)
            def _():
                outg[...] = o

            @pl.when(g == 1)
            def _():
                d = outg[...] - o
                out_ref[...] = jnp.sqrt(jnp.sum(d * d, axis=1))[None, :]

    xspec = pl.BlockSpec((NB, H), lambda g, j: (g * NBLK + j, 0))
    wspec = pl.BlockSpec((H, H), lambda g, j: (0, 0))
    bspec = pl.BlockSpec((1, H), lambda g, j: (0, 0))
    return pl.pallas_call(
        body,
        grid=(NG, NBLK),
        in_specs=[
            xspec, xspec, xspec, xspec,
            pl.BlockSpec((1, 1, NB), lambda g, j: (g * NBLK + j, 0, 0)),
            wspec, bspec, wspec, bspec,
            pl.BlockSpec((4 * H, H), lambda g, j: (0, 0)),
            bspec, wspec, bspec,
        ],
        out_specs=pl.BlockSpec((1, G), lambda g, j: (0, 0)),
        out_shape=jax.ShapeDtypeStruct((1, G), _f32),
        scratch_shapes=[
            pltpu.VMEM((G, 4 * H), _f32),
            pltpu.VMEM((G, H), _f32),
        ],
    )(agg2, x0, x1, x2, batch3d, w1, b1, w2, b2, wp1, bp1, wp2, bp2)


def _prep_edges(edge_index, goff):
    pad = EPAD - E
    src = jnp.concatenate([
        edge_index[0] + goff,
        (jnp.arange(pad, dtype=jnp.int32) * 37) % N + goff,
    ])
    dst = jnp.concatenate([
        edge_index[1],
        N + (jnp.arange(pad, dtype=jnp.int32) % (NLOC - N)),
    ])
    return src.reshape(NS * C, K), dst.reshape(NS * C, K)


def kernel(g_x, g_edge_index, g_batch, h_x, h_edge_index, h_batch,
           W_pre, b_pre, W1, b1, W2, b2, Wp1, bp1, Wp2, bp2):
    X = jnp.concatenate([g_x, h_x], axis=0)
    sg, dg = _prep_edges(g_edge_index, 0)
    sh, dh = _prep_edges(h_edge_index, N)
    src_all = jnp.concatenate([sg, sh], axis=0)
    dst_all = jnp.concatenate([dg, dh], axis=0)

    x0 = _pre_call(X, W_pre, b_pre.reshape(1, H))
    x = x0
    xs = [x0]
    for i in range(2):
        hagg = _agg_call(x, src_all, dst_all)
        x = _mlp_call(hagg, W1[i], b1[i].reshape(1, H),
                      W2[i], b2[i].reshape(1, H),
                      xres=x0 if i == 1 else None)
        xs.append(x)
    hagg2 = _agg_call(x, src_all, dst_all)

    batch3d = jnp.concatenate([g_batch, h_batch]).reshape(NG * NBLK, 1, NB)
    out = _pool_call(hagg2, xs[0], xs[1], xs[2], batch3d,
                     W1[2], b1[2].reshape(1, H), W2[2], b2[2].reshape(1, H),
                     Wp1, bp1.reshape(1, H), Wp2, bp2.reshape(1, H))
    return out.reshape(G)


# speedup vs baseline: 8.2768x; 1.1435x over previous
import functools

import jax
import jax.numpy as jnp
from jax import lax
from jax.experimental import pallas as pl
from jax.experimental.pallas import tpu as pltpu
from jax.experimental.pallas import tpu_sc as plsc

N = 10000
E = 320000
H = 128
G = 64
NG = 2
NC = 2
NS = 16
K = 128
C = 161
TPT = C * K
EPAD = NS * TPT
NLOC = 10048
IPR = 624
REM = N - NS * IPR

NBUF = 3
D = 1
NIB = 6

BM = 800
NB = 400
NBLK = N // NB

_f32 = jnp.float32


def _make_agg():
    mesh = plsc.VectorSubcoreMesh(core_axis_name="c", subcore_axis_name="s",
                                  num_cores=NC, num_subcores=NS)

    @functools.partial(
        pl.kernel,
        out_type=jax.ShapeDtypeStruct((NG * N, H), _f32),
        mesh=mesh,
        scratch_types=(
            [pltpu.VMEM((2, K), jnp.int32) for _ in range(NIB)]
            + [pltpu.VMEM((K, H), _f32) for _ in range(NBUF)]
            + [pltpu.VMEM_SHARED((NLOC, H), _f32)]
            + [pltpu.SemaphoreType.DMA for _ in range(2 * NBUF + NIB)]
        ),
    )
    def agg(x_hbm, sd_hbm, out_hbm, *scratch):
        ibuf = scratch[0:NIB]
        rows = scratch[NIB:NIB + NBUF]
        agg_sh = scratch[NIB + NBUF]
        sems = scratch[NIB + NBUF + 1:]
        gsem = sems[0:NBUF]
        ssem = sems[NBUF:2 * NBUF]
        isem = sems[2 * NBUF:]
        c = lax.axis_index("c")
        s = lax.axis_index("s")
        pltpu.sync_copy(x_hbm.at[pl.ds(c * N + s * IPR, IPR)],
                        agg_sh.at[pl.ds(s * IPR, IPR)])

        @pl.when(s == 0)
        def _():
            pltpu.sync_copy(x_hbm.at[pl.ds(c * N + NS * IPR, REM)],
                            agg_sh.at[pl.ds(NS * IPR, REM)])

        plsc.subcore_barrier()

        row0 = (c * NS + s) * C

        assert (C + D) % NIB == 0
        pltpu.async_copy(sd_hbm.at[row0], ibuf[0], isem[0])

        def body(i, carry):
            for u in range(NIB):
                jj = i * NIB + u
                b = u % NBUF
                q = u
                qn = (u + 1) % NIB
                bd = (b - D) % NBUF
                qd = (u - D) % NIB

                @pl.when(jnp.logical_and(jj >= NBUF, jj < C))
                def _():
                    pltpu.make_async_copy(
                        rows[b], agg_sh.at[ibuf[(u - NBUF) % NIB].at[1]],
                        ssem[b]).wait()

                @pl.when(jj + 1 < C)
                def _():
                    pltpu.async_copy(sd_hbm.at[row0 + jj + 1], ibuf[qn],
                                     isem[qn])

                @pl.when(jj < C)
                def _():
                    pltpu.make_async_copy(sd_hbm.at[row0 + jj], ibuf[q],
                                          isem[q]).wait()
                    pltpu.async_copy(x_hbm.at[ibuf[q].at[0]], rows[b],
                                     gsem[b])

                @pl.when(jnp.logical_and(jj >= D, jj < C + D))
                def _():
                    pltpu.make_async_copy(
                        x_hbm.at[ibuf[qd].at[0]], rows[bd], gsem[bd]).wait()
                    pltpu.async_copy(rows[bd], agg_sh.at[ibuf[qd].at[1]],
                                     ssem[bd], add=True)
            return carry

        lax.fori_loop(0, (C + D) // NIB, body, 0)
        for jj in range(C - NBUF, C):
            pltpu.make_async_copy(rows[jj % NBUF],
                                  agg_sh.at[ibuf[jj % NIB].at[1]],
                                  ssem[jj % NBUF]).wait()
        plsc.subcore_barrier()
        pltpu.sync_copy(agg_sh.at[pl.ds(s * IPR, IPR)],
                        out_hbm.at[pl.ds(c * N + s * IPR, IPR)])

        @pl.when(s == 0)
        def _():
            pltpu.sync_copy(agg_sh.at[pl.ds(NS * IPR, REM)],
                            out_hbm.at[pl.ds(c * N + NS * IPR, REM)])

    return agg


_agg_call = _make_agg()


def _pre_call(x, w, b):
    def body(x_ref, w_ref, b_ref, out_ref):
        out_ref[...] = jnp.dot(x_ref[...], w_ref[...],
                               preferred_element_type=_f32) + b_ref[...]

    m = x.shape[0]
    return pl.pallas_call(
        body,
        grid=(m // BM,),
        in_specs=[
            pl.BlockSpec((BM, H), lambda i: (i, 0)),
            pl.BlockSpec((H, H), lambda i: (0, 0)),
            pl.BlockSpec((1, H), lambda i: (0, 0)),
        ],
        out_specs=pl.BlockSpec((BM, H), lambda i: (i, 0)),
        out_shape=jax.ShapeDtypeStruct((m, H), _f32),
    )(x, w, b)


def _mlp_call(hagg, w1, b1, w2, b2, xres=None):
    has_res = xres is not None

    def body(*refs):
        if has_res:
            h_ref, r_ref, w1r, b1r, w2r, b2r, out_ref = refs
        else:
            h_ref, w1r, b1r, w2r, b2r, out_ref = refs
        t = jnp.dot(h_ref[...], w1r[...], preferred_element_type=_f32) + b1r[...]
        t = jnp.maximum(t, 0.0)
        y = jnp.dot(t, w2r[...], preferred_element_type=_f32) + b2r[...]
        if has_res:
            y = y + r_ref[...]
        out_ref[...] = jnp.maximum(y, 0.0)

    m = hagg.shape[0]
    in_specs = [pl.BlockSpec((BM, H), lambda i: (i, 0))]
    args = [hagg]
    if has_res:
        in_specs.append(pl.BlockSpec((BM, H), lambda i: (i, 0)))
        args.append(xres)
    in_specs += [
        pl.BlockSpec((H, H), lambda i: (0, 0)),
        pl.BlockSpec((1, H), lambda i: (0, 0)),
        pl.BlockSpec((H, H), lambda i: (0, 0)),
        pl.BlockSpec((1, H), lambda i: (0, 0)),
    ]
    args += [w1, b1, w2, b2]
    return pl.pallas_call(
        body,
        grid=(m // BM,),
        in_specs=in_specs,
        out_specs=pl.BlockSpec((BM, H), lambda i: (i, 0)),
        out_shape=jax.ShapeDtypeStruct((m, H), _f32),
    )(*args)


def _pool_call(agg2, x0, x1, x2, batch3d, w1, b1, w2, b2,
               wp1, bp1, wp2, bp2):
    def body(aggr, x0r, x1r, x2r, br, w1r, b1r, w2r, b2r,
             wp1r, bp1r, wp2r, bp2r, out_ref, acc, outg):
        g = pl.program_id(0)
        j = pl.program_id(1)

        @pl.when(j == 0)
        def _():
            acc[...] = jnp.zeros_like(acc)

        t = jnp.dot(aggr[...], w1r[...], preferred_element_type=_f32)
        t = jnp.maximum(t + b1r[...], 0.0)
        x3 = jnp.dot(t, w2r[...], preferred_element_type=_f32) + b2r[...]
        x3 = jnp.maximum(x3, 0.0)

        bvec = br[0, 0, :]
        onehot = (bvec[None, :] ==
                  lax.broadcasted_iota(jnp.int32, (G, NB), 0)).astype(_f32)
        emb = jnp.concatenate(
            [x0r[...], x1r[...], x2r[...], x3], axis=1)
        acc[...] += jnp.dot(onehot, emb, preferred_element_type=_f32)

        @pl.when(j == NBLK - 1)
        def _():
            t = jnp.dot(acc[...], wp1r[...], preferred_element_type=_f32)
            t = jnp.maximum(t + bp1r[...], 0.0)
            o = jnp.dot(t, wp2r[...], preferred_element_type=_f32) + bp2r[...]

            @pl.when(g == 0)
            def _():
                outg[...] = o

            @pl.when(g == 1)
            def _():
                d = outg[...] - o
                out_ref[...] = jnp.sqrt(jnp.sum(d * d, axis=1))[None, :]

    xspec = pl.BlockSpec((NB, H), lambda g, j: (g * NBLK + j, 0))
    wspec = pl.BlockSpec((H, H), lambda g, j: (0, 0))
    bspec = pl.BlockSpec((1, H), lambda g, j: (0, 0))
    return pl.pallas_call(
        body,
        grid=(NG, NBLK),
        in_specs=[
            xspec, xspec, xspec, xspec,
            pl.BlockSpec((1, 1, NB), lambda g, j: (g * NBLK + j, 0, 0)),
            wspec, bspec, wspec, bspec,
            pl.BlockSpec((4 * H, H), lambda g, j: (0, 0)),
            bspec, wspec, bspec,
        ],
        out_specs=pl.BlockSpec((1, G), lambda g, j: (0, 0)),
        out_shape=jax.ShapeDtypeStruct((1, G), _f32),
        scratch_shapes=[
            pltpu.VMEM((G, 4 * H), _f32),
            pltpu.VMEM((G, H), _f32),
        ],
    )(agg2, x0, x1, x2, batch3d, w1, b1, w2, b2, wp1, bp1, wp2, bp2)


def _prep_edges(edge_index, goff):
    pad = EPAD - E
    src = jnp.concatenate([
        edge_index[0] + goff,
        (jnp.arange(pad, dtype=jnp.int32) * 37) % N + goff,
    ])
    dst = jnp.concatenate([
        edge_index[1],
        N + (jnp.arange(pad, dtype=jnp.int32) % (NLOC - N)),
    ])
    return jnp.stack([src.reshape(NS * C, K), dst.reshape(NS * C, K)],
                     axis=1)


def kernel(g_x, g_edge_index, g_batch, h_x, h_edge_index, h_batch,
           W_pre, b_pre, W1, b1, W2, b2, Wp1, bp1, Wp2, bp2):
    X = jnp.concatenate([g_x, h_x], axis=0)
    sd_all = jnp.concatenate(
        [_prep_edges(g_edge_index, 0), _prep_edges(h_edge_index, N)], axis=0)

    x0 = _pre_call(X, W_pre, b_pre.reshape(1, H))
    x = x0
    xs = [x0]
    for i in range(2):
        hagg = _agg_call(x, sd_all)
        x = _mlp_call(hagg, W1[i], b1[i].reshape(1, H),
                      W2[i], b2[i].reshape(1, H),
                      xres=x0 if i == 1 else None)
        xs.append(x)
    hagg2 = _agg_call(x, sd_all)

    batch3d = jnp.concatenate([g_batch, h_batch]).reshape(NG * NBLK, 1, NB)
    out = _pool_call(hagg2, xs[0], xs[1], xs[2], batch3d,
                     W1[2], b1[2].reshape(1, H), W2[2], b2[2].reshape(1, H),
                     Wp1, bp1.reshape(1, H), Wp2, bp2.reshape(1, H))
    return out.reshape(G)


# speedup vs baseline: 8.6757x; 1.0482x over previous
import functools

import jax
import jax.numpy as jnp
from jax import lax
from jax.experimental import pallas as pl
from jax.experimental.pallas import tpu as pltpu
from jax.experimental.pallas import tpu_sc as plsc

N = 10000
E = 320000
H = 128
G = 64
NG = 2
NC = 2
NS = 16
K = 128
C = 160
TPT = C * K
EPAD = NS * TPT
NLOC = 10048
IPR = 624
REM = N - NS * IPR

NBUF = 3
D = 2
NIB = 6

BM = 800
NB = 400
NBLK = N // NB

_f32 = jnp.float32


def _make_agg():
    mesh = plsc.VectorSubcoreMesh(core_axis_name="c", subcore_axis_name="s",
                                  num_cores=NC, num_subcores=NS)

    @functools.partial(
        pl.kernel,
        out_type=jax.ShapeDtypeStruct((NG * N, H), _f32),
        mesh=mesh,
        scratch_types=(
            [pltpu.VMEM((2, K), jnp.int32) for _ in range(NIB)]
            + [pltpu.VMEM((K, H), _f32) for _ in range(NBUF)]
            + [pltpu.VMEM_SHARED((NLOC, H), _f32)]
            + [pltpu.SemaphoreType.DMA for _ in range(2 * NBUF + NIB)]
        ),
    )
    def agg(x_hbm, sd_hbm, out_hbm, *scratch):
        ibuf = scratch[0:NIB]
        rows = scratch[NIB:NIB + NBUF]
        agg_sh = scratch[NIB + NBUF]
        sems = scratch[NIB + NBUF + 1:]
        gsem = sems[0:NBUF]
        ssem = sems[NBUF:2 * NBUF]
        isem = sems[2 * NBUF:]
        c = lax.axis_index("c")
        s = lax.axis_index("s")
        pltpu.sync_copy(x_hbm.at[pl.ds(c * N + s * IPR, IPR)],
                        agg_sh.at[pl.ds(s * IPR, IPR)])

        @pl.when(s == 0)
        def _():
            pltpu.sync_copy(x_hbm.at[pl.ds(c * N + NS * IPR, REM)],
                            agg_sh.at[pl.ds(NS * IPR, REM)])

        plsc.subcore_barrier()

        row0 = (c * NS + s) * C

        assert (C + D) % NIB == 0
        pltpu.async_copy(sd_hbm.at[row0], ibuf[0], isem[0])

        def body(i, carry):
            for u in range(NIB):
                jj = i * NIB + u
                b = u % NBUF
                q = u
                qn = (u + 1) % NIB
                bd = (b - D) % NBUF
                qd = (u - D) % NIB

                @pl.when(jnp.logical_and(jj >= NBUF, jj < C))
                def _():
                    pltpu.make_async_copy(
                        rows[b], agg_sh.at[ibuf[(u - NBUF) % NIB].at[1]],
                        ssem[b]).wait()

                @pl.when(jj + 1 < C)
                def _():
                    pltpu.async_copy(sd_hbm.at[row0 + jj + 1], ibuf[qn],
                                     isem[qn])

                @pl.when(jj < C)
                def _():
                    pltpu.make_async_copy(sd_hbm.at[row0 + jj], ibuf[q],
                                          isem[q]).wait()
                    pltpu.async_copy(x_hbm.at[ibuf[q].at[0]], rows[b],
                                     gsem[b])

                @pl.when(jnp.logical_and(jj >= D, jj < C + D))
                def _():
                    pltpu.make_async_copy(
                        x_hbm.at[ibuf[qd].at[0]], rows[bd], gsem[bd]).wait()
                    pltpu.async_copy(rows[bd], agg_sh.at[ibuf[qd].at[1]],
                                     ssem[bd], add=True)
            return carry

        lax.fori_loop(0, (C + D) // NIB, body, 0)
        for jj in range(C - NBUF, C):
            pltpu.make_async_copy(rows[jj % NBUF],
                                  agg_sh.at[ibuf[jj % NIB].at[1]],
                                  ssem[jj % NBUF]).wait()
        plsc.subcore_barrier()
        pltpu.sync_copy(agg_sh.at[pl.ds(s * IPR, IPR)],
                        out_hbm.at[pl.ds(c * N + s * IPR, IPR)])

        @pl.when(s == 0)
        def _():
            pltpu.sync_copy(agg_sh.at[pl.ds(NS * IPR, REM)],
                            out_hbm.at[pl.ds(c * N + NS * IPR, REM)])

    return agg


_agg_call = _make_agg()


def _pre_call(x, w, b):
    def body(x_ref, w_ref, b_ref, out_ref):
        out_ref[...] = jnp.dot(x_ref[...], w_ref[...],
                               preferred_element_type=_f32) + b_ref[...]

    m = x.shape[0]
    return pl.pallas_call(
        body,
        grid=(m // BM,),
        in_specs=[
            pl.BlockSpec((BM, H), lambda i: (i, 0)),
            pl.BlockSpec((H, H), lambda i: (0, 0)),
            pl.BlockSpec((1, H), lambda i: (0, 0)),
        ],
        out_specs=pl.BlockSpec((BM, H), lambda i: (i, 0)),
        out_shape=jax.ShapeDtypeStruct((m, H), _f32),
    )(x, w, b)


def _mlp_call(hagg, w1, b1, w2, b2, xres=None):
    has_res = xres is not None

    def body(*refs):
        if has_res:
            h_ref, r_ref, w1r, b1r, w2r, b2r, out_ref = refs
        else:
            h_ref, w1r, b1r, w2r, b2r, out_ref = refs
        t = jnp.dot(h_ref[...], w1r[...], preferred_element_type=_f32) + b1r[...]
        t = jnp.maximum(t, 0.0)
        y = jnp.dot(t, w2r[...], preferred_element_type=_f32) + b2r[...]
        if has_res:
            y = y + r_ref[...]
        out_ref[...] = jnp.maximum(y, 0.0)

    m = hagg.shape[0]
    in_specs = [pl.BlockSpec((BM, H), lambda i: (i, 0))]
    args = [hagg]
    if has_res:
        in_specs.append(pl.BlockSpec((BM, H), lambda i: (i, 0)))
        args.append(xres)
    in_specs += [
        pl.BlockSpec((H, H), lambda i: (0, 0)),
        pl.BlockSpec((1, H), lambda i: (0, 0)),
        pl.BlockSpec((H, H), lambda i: (0, 0)),
        pl.BlockSpec((1, H), lambda i: (0, 0)),
    ]
    args += [w1, b1, w2, b2]
    return pl.pallas_call(
        body,
        grid=(m // BM,),
        in_specs=in_specs,
        out_specs=pl.BlockSpec((BM, H), lambda i: (i, 0)),
        out_shape=jax.ShapeDtypeStruct((m, H), _f32),
    )(*args)


def _pool_call(agg2, x0, x1, x2, batch3d, w1, b1, w2, b2,
               wp1, bp1, wp2, bp2):
    def body(aggr, x0r, x1r, x2r, br, w1r, b1r, w2r, b2r,
             wp1r, bp1r, wp2r, bp2r, out_ref, acc, outg):
        g = pl.program_id(0)
        j = pl.program_id(1)

        @pl.when(j == 0)
        def _():
            acc[...] = jnp.zeros_like(acc)

        t = jnp.dot(aggr[...], w1r[...], preferred_element_type=_f32)
        t = jnp.maximum(t + b1r[...], 0.0)
        x3 = jnp.dot(t, w2r[...], preferred_element_type=_f32) + b2r[...]
        x3 = jnp.maximum(x3, 0.0)

        bvec = br[0, 0, :]
        onehot = (bvec[None, :] ==
                  lax.broadcasted_iota(jnp.int32, (G, NB), 0)).astype(_f32)
        emb = jnp.concatenate(
            [x0r[...], x1r[...], x2r[...], x3], axis=1)
        acc[...] += jnp.dot(onehot, emb, preferred_element_type=_f32)

        @pl.when(j == NBLK - 1)
        def _():
            t = jnp.dot(acc[...], wp1r[...], preferred_element_type=_f32)
            t = jnp.maximum(t + bp1r[...], 0.0)
            o = jnp.dot(t, wp2r[...], preferred_element_type=_f32) + bp2r[...]

            @pl.when(g == 0)
            def _():
                outg[...] = o

            @pl.when(g == 1)
            def _():
                d = outg[...] - o
                out_ref[...] = jnp.sqrt(jnp.sum(d * d, axis=1))[None, :]

    xspec = pl.BlockSpec((NB, H), lambda g, j: (g * NBLK + j, 0))
    wspec = pl.BlockSpec((H, H), lambda g, j: (0, 0))
    bspec = pl.BlockSpec((1, H), lambda g, j: (0, 0))
    return pl.pallas_call(
        body,
        grid=(NG, NBLK),
        in_specs=[
            xspec, xspec, xspec, xspec,
            pl.BlockSpec((1, 1, NB), lambda g, j: (g * NBLK + j, 0, 0)),
            wspec, bspec, wspec, bspec,
            pl.BlockSpec((4 * H, H), lambda g, j: (0, 0)),
            bspec, wspec, bspec,
        ],
        out_specs=pl.BlockSpec((1, G), lambda g, j: (0, 0)),
        out_shape=jax.ShapeDtypeStruct((1, G), _f32),
        scratch_shapes=[
            pltpu.VMEM((G, 4 * H), _f32),
            pltpu.VMEM((G, H), _f32),
        ],
    )(agg2, x0, x1, x2, batch3d, w1, b1, w2, b2, wp1, bp1, wp2, bp2)


def _prep_edges(edge_index, goff):
    pad = EPAD - E
    src = jnp.concatenate([
        edge_index[0] + goff,
        (jnp.arange(pad, dtype=jnp.int32) * 37) % N + goff,
    ])
    dst = jnp.concatenate([
        edge_index[1],
        N + (jnp.arange(pad, dtype=jnp.int32) % (NLOC - N)),
    ])
    return jnp.stack([src.reshape(NS * C, K), dst.reshape(NS * C, K)],
                     axis=1)


def kernel(g_x, g_edge_index, g_batch, h_x, h_edge_index, h_batch,
           W_pre, b_pre, W1, b1, W2, b2, Wp1, bp1, Wp2, bp2):
    X = jnp.concatenate([g_x, h_x], axis=0)
    sd_all = jnp.concatenate(
        [_prep_edges(g_edge_index, 0), _prep_edges(h_edge_index, N)], axis=0)

    x0 = _pre_call(X, W_pre, b_pre.reshape(1, H))
    x = x0
    xs = [x0]
    for i in range(2):
        hagg = _agg_call(x, sd_all)
        x = _mlp_call(hagg, W1[i], b1[i].reshape(1, H),
                      W2[i], b2[i].reshape(1, H),
                      xres=x0 if i == 1 else None)
        xs.append(x)
    hagg2 = _agg_call(x, sd_all)

    batch3d = jnp.concatenate([g_batch, h_batch]).reshape(NG * NBLK, 1, NB)
    out = _pool_call(hagg2, xs[0], xs[1], xs[2], batch3d,
                     W1[2], b1[2].reshape(1, H), W2[2], b2[2].reshape(1, H),
                     Wp1, bp1.reshape(1, H), Wp2, bp2.reshape(1, H))
    return out.reshape(G)


# speedup vs baseline: 8.9852x; 1.0357x over previous
import functools

import jax
import jax.numpy as jnp
from jax import lax
from jax.experimental import pallas as pl
from jax.experimental.pallas import tpu as pltpu
from jax.experimental.pallas import tpu_sc as plsc

N = 10000
E = 320000
H = 128
G = 64
NC = 2
NS = 16
NW = NC * NS
K = 128
C = 79
EPAD = NW * C * K
NLOC = 10048
IPR = 624
REM = N - NS * IPR

NBUF = 3
D = 2
NIB = 6

BM = 1000
NB = 400
NBLK = N // NB

_f32 = jnp.float32


def _make_agg():
    mesh = plsc.VectorSubcoreMesh(core_axis_name="c", subcore_axis_name="s",
                                  num_cores=NC, num_subcores=NS)

    @functools.partial(
        pl.kernel,
        out_type=jax.ShapeDtypeStruct((NC * N, H), _f32),
        mesh=mesh,
        scratch_types=(
            [pltpu.VMEM((2, K), jnp.int32) for _ in range(NIB)]
            + [pltpu.VMEM((K, H), _f32) for _ in range(NBUF)]
            + [pltpu.VMEM_SHARED((NLOC, H), _f32)]
            + [pltpu.SemaphoreType.DMA for _ in range(2 * NBUF + NIB)]
        ),
    )
    def agg(x_hbm, sd_hbm, z_hbm, out_hbm, *scratch):
        ibuf = scratch[0:NIB]
        rows = scratch[NIB:NIB + NBUF]
        agg_sh = scratch[NIB + NBUF]
        sems = scratch[NIB + NBUF + 1:]
        gsem = sems[0:NBUF]
        ssem = sems[NBUF:2 * NBUF]
        isem = sems[2 * NBUF:]
        c = lax.axis_index("c")
        s = lax.axis_index("s")

        @pl.when(c == 0)
        def _():
            pltpu.sync_copy(x_hbm.at[pl.ds(s * IPR, IPR)],
                            agg_sh.at[pl.ds(s * IPR, IPR)])

            @pl.when(s == 0)
            def _():
                pltpu.sync_copy(x_hbm.at[pl.ds(NS * IPR, REM)],
                                agg_sh.at[pl.ds(NS * IPR, REM)])

        @pl.when(c == 1)
        def _():
            pltpu.sync_copy(z_hbm.at[pl.ds(0, IPR)],
                            agg_sh.at[pl.ds(s * IPR, IPR)])

            @pl.when(s == 0)
            def _():
                pltpu.sync_copy(z_hbm.at[pl.ds(0, REM)],
                                agg_sh.at[pl.ds(NS * IPR, REM)])

        plsc.subcore_barrier()

        row0 = (c * NS + s) * C

        pltpu.async_copy(sd_hbm.at[row0], ibuf[0], isem[0])

        def body(i, carry):
            for u in range(NIB):
                jj = i * NIB + u
                b = u % NBUF
                q = u
                qn = (u + 1) % NIB
                bd = (b - D) % NBUF
                qd = (u - D) % NIB

                @pl.when(jnp.logical_and(jj >= NBUF, jj < C))
                def _():
                    pltpu.make_async_copy(
                        rows[b], agg_sh.at[ibuf[(u - NBUF) % NIB].at[1]],
                        ssem[b]).wait()

                @pl.when(jj + 1 < C)
                def _():
                    pltpu.async_copy(sd_hbm.at[row0 + jj + 1], ibuf[qn],
                                     isem[qn])

                @pl.when(jj < C)
                def _():
                    pltpu.make_async_copy(sd_hbm.at[row0 + jj], ibuf[q],
                                          isem[q]).wait()
                    pltpu.async_copy(x_hbm.at[ibuf[q].at[0]], rows[b],
                                     gsem[b])

                @pl.when(jnp.logical_and(jj >= D, jj < C + D))
                def _():
                    pltpu.make_async_copy(
                        x_hbm.at[ibuf[qd].at[0]], rows[bd], gsem[bd]).wait()
                    pltpu.async_copy(rows[bd], agg_sh.at[ibuf[qd].at[1]],
                                     ssem[bd], add=True)
            return carry

        lax.fori_loop(0, (C + D + NIB - 1) // NIB, body, 0)
        for jj in range(C - NBUF, C):
            pltpu.make_async_copy(rows[jj % NBUF],
                                  agg_sh.at[ibuf[jj % NIB].at[1]],
                                  ssem[jj % NBUF]).wait()
        plsc.subcore_barrier()
        pltpu.sync_copy(agg_sh.at[pl.ds(s * IPR, IPR)],
                        out_hbm.at[pl.ds(c * N + s * IPR, IPR)])

        @pl.when(s == 0)
        def _():
            pltpu.sync_copy(agg_sh.at[pl.ds(NS * IPR, REM)],
                            out_hbm.at[pl.ds(c * N + NS * IPR, REM)])

    return agg


_agg_call = _make_agg()


def _pre_call(x, w, b):
    def body(x_ref, w_ref, b_ref, out_ref):
        out_ref[...] = jnp.dot(x_ref[...], w_ref[...],
                               preferred_element_type=_f32) + b_ref[...]

    return pl.pallas_call(
        body,
        grid=(N // BM,),
        in_specs=[
            pl.BlockSpec((BM, H), lambda i: (i, 0)),
            pl.BlockSpec((H, H), lambda i: (0, 0)),
            pl.BlockSpec((1, H), lambda i: (0, 0)),
        ],
        out_specs=pl.BlockSpec((BM, H), lambda i: (i, 0)),
        out_shape=jax.ShapeDtypeStruct((N, H), _f32),
    )(x, w, b)


def _mlp_call(parts, w1, b1, w2, b2, xres=None):
    has_res = xres is not None
    nblk = N // BM

    def body(*refs):
        if has_res:
            p0_ref, p1_ref, r_ref, w1r, b1r, w2r, b2r, out_ref = refs
        else:
            p0_ref, p1_ref, w1r, b1r, w2r, b2r, out_ref = refs
        h = p0_ref[...] + p1_ref[...]
        t = jnp.dot(h, w1r[...], preferred_element_type=_f32) + b1r[...]
        t = jnp.maximum(t, 0.0)
        y = jnp.dot(t, w2r[...], preferred_element_type=_f32) + b2r[...]
        if has_res:
            y = y + r_ref[...]
        out_ref[...] = jnp.maximum(y, 0.0)

    in_specs = [
        pl.BlockSpec((BM, H), lambda i: (i, 0)),
        pl.BlockSpec((BM, H), lambda i: (nblk + i, 0)),
    ]
    args = [parts, parts]
    if has_res:
        in_specs.append(pl.BlockSpec((BM, H), lambda i: (i, 0)))
        args.append(xres)
    in_specs += [
        pl.BlockSpec((H, H), lambda i: (0, 0)),
        pl.BlockSpec((1, H), lambda i: (0, 0)),
        pl.BlockSpec((H, H), lambda i: (0, 0)),
        pl.BlockSpec((1, H), lambda i: (0, 0)),
    ]
    args += [w1, b1, w2, b2]
    return pl.pallas_call(
        body,
        grid=(nblk,),
        in_specs=in_specs,
        out_specs=pl.BlockSpec((BM, H), lambda i: (i, 0)),
        out_shape=jax.ShapeDtypeStruct((N, H), _f32),
    )(*args)


def _pool_call(parts2, x0, x1, x2, batch3d, w1, b1, w2, b2):
    def body(p0r, p1r, x0r, x1r, x2r, br, w1r, b1r, w2r, b2r, out_ref, acc):
        j = pl.program_id(0)

        @pl.when(j == 0)
        def _():
            acc[...] = jnp.zeros_like(acc)

        t = jnp.dot(p0r[...] + p1r[...], w1r[...],
                    preferred_element_type=_f32)
        t = jnp.maximum(t + b1r[...], 0.0)
        x3 = jnp.dot(t, w2r[...], preferred_element_type=_f32) + b2r[...]
        x3 = jnp.maximum(x3, 0.0)

        bvec = br[0, 0, :]
        onehot = (bvec[None, :] ==
                  lax.broadcasted_iota(jnp.int32, (G, NB), 0)).astype(_f32)
        emb = jnp.concatenate([x0r[...], x1r[...], x2r[...], x3], axis=1)
        acc[...] += jnp.dot(onehot, emb, preferred_element_type=_f32)

        @pl.when(j == NBLK - 1)
        def _():
            out_ref[...] = acc[...]

    xspec = pl.BlockSpec((NB, H), lambda j: (j, 0))
    wspec = pl.BlockSpec((H, H), lambda j: (0, 0))
    bspec = pl.BlockSpec((1, H), lambda j: (0, 0))
    return pl.pallas_call(
        body,
        grid=(NBLK,),
        in_specs=[
            pl.BlockSpec((NB, H), lambda j: (j, 0)),
            pl.BlockSpec((NB, H), lambda j: (NBLK + j, 0)),
            xspec, xspec, xspec,
            pl.BlockSpec((1, 1, NB), lambda j: (j, 0, 0)),
            wspec, bspec, wspec, bspec,
        ],
        out_specs=pl.BlockSpec((G, 4 * H), lambda j: (0, 0)),
        out_shape=jax.ShapeDtypeStruct((G, 4 * H), _f32),
        scratch_shapes=[pltpu.VMEM((G, 4 * H), _f32)],
    )(parts2, parts2, x0, x1, x2, batch3d, w1, b1, w2, b2)


def _final_call(pg, ph, wp1, bp1, wp2, bp2):
    def body(pgr, phr, wp1r, bp1r, wp2r, bp2r, out_ref):
        def head(p):
            t = jnp.dot(p, wp1r[...], preferred_element_type=_f32)
            t = jnp.maximum(t + bp1r[...], 0.0)
            return jnp.dot(t, wp2r[...],
                           preferred_element_type=_f32) + bp2r[...]

        d = head(pgr[...]) - head(phr[...])
        out_ref[...] = jnp.sqrt(jnp.sum(d * d, axis=1))[None, :]

    return pl.pallas_call(
        body,
        out_shape=jax.ShapeDtypeStruct((1, G), _f32),
    )(pg, ph, wp1, bp1, wp2, bp2)


def _prep_edges(edge_index):
    pad = EPAD - E
    src = jnp.concatenate([
        edge_index[0],
        (jnp.arange(pad, dtype=jnp.int32) * 37) % N,
    ])
    dst = jnp.concatenate([
        edge_index[1],
        N + (jnp.arange(pad, dtype=jnp.int32) % (NLOC - N)),
    ])
    return jnp.stack([src.reshape(NW * C, K), dst.reshape(NW * C, K)],
                     axis=1)


def kernel(g_x, g_edge_index, g_batch, h_x, h_edge_index, h_batch,
           W_pre, b_pre, W1, b1, W2, b2, Wp1, bp1, Wp2, bp2):
    sd_g = _prep_edges(g_edge_index)
    sd_h = _prep_edges(h_edge_index)
    zer = jnp.zeros((IPR, H), _f32)
    b_pre2 = b_pre.reshape(1, H)
    b1r = [b1[i].reshape(1, H) for i in range(3)]
    b2r = [b2[i].reshape(1, H) for i in range(3)]

    x0g = _pre_call(g_x, W_pre, b_pre2)
    x0h = _pre_call(h_x, W_pre, b_pre2)
    xg, xh = x0g, x0h
    xsg, xsh = [x0g], [x0h]
    for i in range(2):
        pg = _agg_call(xg, sd_g, zer)
        ph = _agg_call(xh, sd_h, zer)
        xg = _mlp_call(pg, W1[i], b1r[i], W2[i], b2r[i],
                       xres=x0g if i == 1 else None)
        xh = _mlp_call(ph, W1[i], b1r[i], W2[i], b2r[i],
                       xres=x0h if i == 1 else None)
        xsg.append(xg)
        xsh.append(xh)
    pg2 = _agg_call(xg, sd_g, zer)
    ph2 = _agg_call(xh, sd_h, zer)

    plg = _pool_call(pg2, xsg[0], xsg[1], xsg[2],
                     g_batch.reshape(NBLK, 1, NB),
                     W1[2], b1r[2], W2[2], b2r[2])
    plh = _pool_call(ph2, xsh[0], xsh[1], xsh[2],
                     h_batch.reshape(NBLK, 1, NB),
                     W1[2], b1r[2], W2[2], b2r[2])
    out = _final_call(plg, plh, Wp1, bp1.reshape(1, H),
                      Wp2, bp2.reshape(1, H))
    return out.reshape(G)
